# baseline (speedup 1.0000x reference)
import functools

import jax
import jax.numpy as jnp
from jax import lax
from jax.experimental import pallas as pl
from jax.experimental.pallas import tpu as pltpu
from jax.experimental.pallas import tpu_sc as plsc

B = 4
N = 16384
M = 1024
S = 32
R2 = 0.25
NW = 32
CH = 32
CW = 128


def _fps_body(x_ref, out_ref):
    ii = (lax.broadcasted_iota(jnp.int32, (128, 128), 0) * 128
          + lax.broadcasted_iota(jnp.int32, (128, 128), 1))
    xs = [[x_ref[b, c] for c in range(3)] for b in range(B)]

    def body(i, st):
        dists, fs = st
        nd, nf = [], []
        for b in range(B):
            x0, x1, x2 = xs[b]
            sel = ii == fs[b]
            c0 = jnp.sum(jnp.where(sel, x0, 0.0))
            c1 = jnp.sum(jnp.where(sel, x1, 0.0))
            c2 = jnp.sum(jnp.where(sel, x2, 0.0))
            out_ref[b, 0, i] = c0
            out_ref[b, 1, i] = c1
            out_ref[b, 2, i] = c2
            d = (x0 - c0) ** 2 + (x1 - c1) ** 2 + (x2 - c2) ** 2
            db = jnp.minimum(dists[b], d)
            m = jnp.max(db)
            nf.append(jnp.min(jnp.where(db == m, ii, jnp.int32(N))))
            nd.append(db)
        return tuple(nd), tuple(nf)

    lax.fori_loop(
        0, M, body,
        (tuple(jnp.full((128, 128), 1e10, jnp.float32) for _ in range(B)),
         tuple(jnp.int32(0) for _ in range(B))))


def _fps(xc):
    out = pl.pallas_call(
        _fps_body,
        out_shape=jax.ShapeDtypeStruct((B, 3, M), jnp.float32),
        out_specs=pl.BlockSpec(memory_space=pltpu.SMEM),
    )(xc)
    return jnp.transpose(out, (0, 2, 1))


def _bq_body(xt_ref, q_ref, idx_ref, nm_ref, d_ref, cm_ref):
    b = pl.program_id(0)
    k = pl.program_id(2)
    base = b * N

    @pl.when(k == 0)
    def _init():
        X = xt_ref[0]
        Q = q_ref[0]
        x2 = jnp.sum(X * X, axis=0, keepdims=True)
        q2 = jnp.sum(Q * Q, axis=1, keepdims=True)
        dot = jnp.dot(Q, X, preferred_element_type=jnp.float32)
        dist2 = (q2 + x2) - 2.0 * dot
        Dm = jnp.where(dist2 <= R2, dist2, jnp.inf)
        d_ref[...] = Dm
        cm_ref[...] = jnp.min(Dm.reshape(128, 128, 128), axis=2)

    D = d_ref[...]
    ii = lax.broadcasted_iota(jnp.int32, (128, N), 1)
    mv = jnp.min(cm_ref[...], axis=1, keepdims=True)
    pos = jnp.min(jnp.where(D == mv, ii, jnp.int32(N)), axis=1)
    valid = mv[:, 0] <= R2

    @pl.when(k == 0)
    def _nm():
        nm_ref[0] = jnp.broadcast_to(
            valid.astype(jnp.float32)[:, None], (128, 8))

    fb = jnp.where(k == 0, jnp.full((128,), base, jnp.int32),
                   idx_ref[0, :, 0])
    sel = jnp.where(valid, pos + base, fb)
    jj = lax.broadcasted_iota(jnp.int32, (128, S), 1)
    idx_ref[0] = jnp.where(jj == k, sel[:, None], idx_ref[0])
    Du = jnp.where(ii == pos[:, None], jnp.inf, D)
    d_ref[...] = Du
    cm_ref[...] = jnp.min(Du.reshape(128, 128, 128), axis=2)


def _ball_query(xt, new_xyz):
    return pl.pallas_call(
        _bq_body,
        grid=(B, M // 128, S),
        in_specs=[
            pl.BlockSpec((1, 3, N), lambda b, m, k: (b, 0, 0)),
            pl.BlockSpec((1, 128, 3), lambda b, m, k: (b, m, 0)),
        ],
        out_specs=[
            pl.BlockSpec((1, 128, S), lambda b, m, k: (b, m, 0)),
            pl.BlockSpec((1, 128, 8), lambda b, m, k: (b, m, 0)),
        ],
        out_shape=[
            jax.ShapeDtypeStruct((B, M, S), jnp.int32),
            jax.ShapeDtypeStruct((B, M, 8), jnp.float32),
        ],
        scratch_shapes=[pltpu.VMEM((128, N), jnp.float32),
                        pltpu.VMEM((128, 128), jnp.float32)],
    )(xt, new_xyz)


def _t_body(x_ref, f_ref, wx_ref, wf_ref, t_ref):
    t_ref[0] = (
        jnp.dot(x_ref[0], wx_ref[...], preferred_element_type=jnp.float32)
        + jnp.dot(f_ref[0], wf_ref[...], preferred_element_type=jnp.float32))


def _pre_transform(xyz, ft, wxt, wft):
    return pl.pallas_call(
        _t_body,
        grid=(B,),
        in_specs=[
            pl.BlockSpec((1, N, 3), lambda b: (b, 0, 0)),
            pl.BlockSpec((1, N, 64), lambda b: (b, 0, 0)),
            pl.BlockSpec((3, 64), lambda b: (0, 0)),
            pl.BlockSpec((64, 64), lambda b: (0, 0)),
        ],
        out_specs=pl.BlockSpec((1, N, 64), lambda b: (b, 0, 0)),
        out_shape=jax.ShapeDtypeStruct((B, N, 64), jnp.float32),
    )(xyz, ft, wxt, wft)


def _sc_gather_body(idx_hbm, tab_hbm, out_hbm, idx_v, rows_v, sem):
    wid = lax.axis_index("s") * 2 + lax.axis_index("c")
    pltpu.sync_copy(idx_hbm.at[wid], idx_v)

    def body(j, carry):
        pltpu.async_copy(tab_hbm.at[idx_v.at[j]], rows_v, sem).wait()
        pltpu.sync_copy(rows_v, out_hbm.at[wid * CH + j])
        return carry

    lax.fori_loop(0, CH, body, 0)


def _gather_rows(idx3, tab):
    mesh = plsc.VectorSubcoreMesh(core_axis_name="c", subcore_axis_name="s")
    fn = pl.kernel(
        _sc_gather_body,
        mesh=mesh,
        compiler_params=pltpu.CompilerParams(use_tc_tiling_on_sc=False),
        out_type=jax.ShapeDtypeStruct((NW * CH, CW, 64), jnp.float32),
        scratch_types=[
            pltpu.VMEM((CH, CW), jnp.int32),
            pltpu.VMEM((CW, 64), jnp.float32),
            pltpu.SemaphoreType.DMA,
        ],
    )
    return fn(idx3, tab)


def _mlp_body(g_ref, q_ref, nm_ref, wx_ref, b0_ref, w1_ref, b1_ref,
              w2_ref, b2_ref, out_ref):
    g = g_ref[0]
    qb = q_ref[0]
    u = jnp.dot(qb, wx_ref[...], preferred_element_type=jnp.float32)
    h = jnp.maximum(g - u[:, None, :] + b0_ref[...][None, None, :], 0.0)
    h = h.reshape(256 * S, 64)
    h = jnp.maximum(
        jnp.dot(h, w1_ref[...], preferred_element_type=jnp.float32)
        + b1_ref[...][None, :], 0.0)
    h = jnp.maximum(
        jnp.dot(h, w2_ref[...], preferred_element_type=jnp.float32)
        + b2_ref[...][None, :], 0.0)
    h = h.reshape(256, S, 128)
    nm = nm_ref[0][:, 0]
    out_ref[0] = jnp.max(h, axis=1) * nm[:, None]


def _mlp_pool(g4, new_xyz, nm, wxt, b0, w1t, b1, w2t, b2):
    return pl.pallas_call(
        _mlp_body,
        grid=(B, M // 256),
        in_specs=[
            pl.BlockSpec((1, 256, S, 64), lambda b, m: (b, m, 0, 0)),
            pl.BlockSpec((1, 256, 3), lambda b, m: (b, m, 0)),
            pl.BlockSpec((1, 256, 8), lambda b, m: (b, m, 0)),
            pl.BlockSpec((3, 64), lambda b, m: (0, 0)),
            pl.BlockSpec((64,), lambda b, m: (0,)),
            pl.BlockSpec((64, 64), lambda b, m: (0, 0)),
            pl.BlockSpec((64,), lambda b, m: (0,)),
            pl.BlockSpec((64, 128), lambda b, m: (0, 0)),
            pl.BlockSpec((128,), lambda b, m: (0,)),
        ],
        out_specs=pl.BlockSpec((1, 256, 128), lambda b, m: (b, m, 0)),
        out_shape=jax.ShapeDtypeStruct((B, M, 128), jnp.float32),
    )(g4, new_xyz, nm, wxt, b0, w1t, b1, w2t, b2)


def kernel(xyz, features, w0, b0, w1, b1, w2, b2):
    xt = jnp.transpose(xyz, (0, 2, 1))
    xc = xt.reshape(B, 3, 128, 128)
    new_xyz = _fps(xc)

    idx_g, nm = _ball_query(xt, new_xyz)

    ft = jnp.transpose(features, (0, 2, 1))
    wxt = jnp.transpose(w0[:, :3])
    wft = jnp.transpose(w0[:, 3:])
    t = _pre_transform(xyz, ft, wxt, wft)

    idx3 = idx_g.reshape(NW, CH, CW)
    g = _gather_rows(idx3, t.reshape(B * N, 64))
    g4 = g.reshape(B, M, S, 64)

    pooled = _mlp_pool(g4, new_xyz, nm, wxt, b0,
                       jnp.transpose(w1), b1, jnp.transpose(w2), b2)
    return new_xyz, jnp.transpose(pooled, (0, 2, 1))


# speedup vs baseline: 1.3287x; 1.3287x over previous
import functools

import jax
import jax.numpy as jnp
from jax import lax
from jax.experimental import pallas as pl
from jax.experimental.pallas import tpu as pltpu
from jax.experimental.pallas import tpu_sc as plsc

B = 4
N = 16384
M = 1024
S = 32
R2 = 0.25
NW = 32
CH = 32
CW = 128


def _fps_body(x_ref, out_ref):
    ii = (lax.broadcasted_iota(jnp.int32, (128, 128), 0) * 128
          + lax.broadcasted_iota(jnp.int32, (128, 128), 1))
    xs = [[x_ref[b, c] for c in range(3)] for b in range(B)]

    def body(i, st):
        dists, fs = st
        nd, nf = [], []
        for b in range(B):
            x0, x1, x2 = xs[b]
            sel = ii == fs[b]
            c0 = jnp.sum(jnp.where(sel, x0, 0.0))
            c1 = jnp.sum(jnp.where(sel, x1, 0.0))
            c2 = jnp.sum(jnp.where(sel, x2, 0.0))
            out_ref[b, 0, i] = c0
            out_ref[b, 1, i] = c1
            out_ref[b, 2, i] = c2
            d = (x0 - c0) ** 2 + (x1 - c1) ** 2 + (x2 - c2) ** 2
            db = jnp.minimum(dists[b], d)
            m = jnp.max(db)
            nf.append(jnp.min(jnp.where(db == m, ii, jnp.int32(N))))
            nd.append(db)
        return tuple(nd), tuple(nf)

    lax.fori_loop(
        0, M, body,
        (tuple(jnp.full((128, 128), 1e10, jnp.float32) for _ in range(B)),
         tuple(jnp.int32(0) for _ in range(B))))


def _fps(xc):
    out = pl.pallas_call(
        _fps_body,
        out_shape=jax.ShapeDtypeStruct((B, 3, M), jnp.float32),
        out_specs=pl.BlockSpec(memory_space=pltpu.SMEM),
    )(xc)
    return jnp.transpose(out, (0, 2, 1))


def _bq_body(xt_ref, q_ref, idx_ref, nm_ref, d_ref):
    b = pl.program_id(0)
    k = pl.program_id(2)
    base = b * N

    @pl.when(k == 0)
    def _init():
        X = xt_ref[0]
        Q = q_ref[0]
        x2 = jnp.sum(X * X, axis=0, keepdims=True)
        q2 = jnp.sum(Q * Q, axis=1, keepdims=True)
        dot = jnp.dot(Q, X, preferred_element_type=jnp.float32)
        dist2 = (q2 + x2) - 2.0 * dot
        d_ref[...] = jnp.where(dist2 <= R2, dist2, jnp.inf)

    D = d_ref[...]
    ii = lax.broadcasted_iota(jnp.int32, (128, N), 1)
    mv = jnp.min(D, axis=1, keepdims=True)
    pos = jnp.min(jnp.where(D == mv, ii, jnp.int32(N)), axis=1)
    valid = mv[:, 0] <= R2

    @pl.when(k == 0)
    def _nm():
        nm_ref[0] = jnp.broadcast_to(
            valid.astype(jnp.float32)[:, None], (128, 8))

    fb = jnp.where(k == 0, jnp.full((128,), base, jnp.int32),
                   idx_ref[0, :, 0])
    sel = jnp.where(valid, pos + base, fb)
    jj = lax.broadcasted_iota(jnp.int32, (128, S), 1)
    idx_ref[0] = jnp.where(jj == k, sel[:, None], idx_ref[0])
    d_ref[...] = jnp.where(ii == pos[:, None], jnp.inf, D)


def _ball_query(xt, new_xyz):
    return pl.pallas_call(
        _bq_body,
        grid=(B, M // 128, S),
        in_specs=[
            pl.BlockSpec((1, 3, N), lambda b, m, k: (b, 0, 0)),
            pl.BlockSpec((1, 128, 3), lambda b, m, k: (b, m, 0)),
        ],
        out_specs=[
            pl.BlockSpec((1, 128, S), lambda b, m, k: (b, m, 0)),
            pl.BlockSpec((1, 128, 8), lambda b, m, k: (b, m, 0)),
        ],
        out_shape=[
            jax.ShapeDtypeStruct((B, M, S), jnp.int32),
            jax.ShapeDtypeStruct((B, M, 8), jnp.float32),
        ],
        scratch_shapes=[pltpu.VMEM((128, N), jnp.float32)],
    )(xt, new_xyz)


def _t_body(x_ref, f_ref, wx_ref, wf_ref, t_ref):
    t_ref[0] = (
        jnp.dot(x_ref[0], wx_ref[...], preferred_element_type=jnp.float32)
        + jnp.dot(f_ref[0], wf_ref[...], preferred_element_type=jnp.float32))


def _pre_transform(xyz, ft, wxt, wft):
    return pl.pallas_call(
        _t_body,
        grid=(B,),
        in_specs=[
            pl.BlockSpec((1, N, 3), lambda b: (b, 0, 0)),
            pl.BlockSpec((1, N, 64), lambda b: (b, 0, 0)),
            pl.BlockSpec((3, 64), lambda b: (0, 0)),
            pl.BlockSpec((64, 64), lambda b: (0, 0)),
        ],
        out_specs=pl.BlockSpec((1, N, 64), lambda b: (b, 0, 0)),
        out_shape=jax.ShapeDtypeStruct((B, N, 64), jnp.float32),
    )(xyz, ft, wxt, wft)


def _sc_gather_body(idx_hbm, tab_hbm, out_hbm, idx_v, rows_v, sem):
    wid = lax.axis_index("s") * 2 + lax.axis_index("c")
    pltpu.sync_copy(idx_hbm.at[wid], idx_v)

    def body(j, carry):
        pltpu.async_copy(tab_hbm.at[idx_v.at[j]], rows_v, sem).wait()
        pltpu.sync_copy(rows_v, out_hbm.at[wid * CH + j])
        return carry

    lax.fori_loop(0, CH, body, 0)


def _gather_rows(idx3, tab):
    mesh = plsc.VectorSubcoreMesh(core_axis_name="c", subcore_axis_name="s")
    fn = pl.kernel(
        _sc_gather_body,
        mesh=mesh,
        compiler_params=pltpu.CompilerParams(use_tc_tiling_on_sc=False),
        out_type=jax.ShapeDtypeStruct((NW * CH, CW, 64), jnp.float32),
        scratch_types=[
            pltpu.VMEM((CH, CW), jnp.int32),
            pltpu.VMEM((CW, 64), jnp.float32),
            pltpu.SemaphoreType.DMA,
        ],
    )
    return fn(idx3, tab)


def _mlp_body(g_ref, q_ref, nm_ref, wx_ref, b0_ref, w1_ref, b1_ref,
              w2_ref, b2_ref, out_ref):
    g = g_ref[0]
    qb = q_ref[0]
    u = jnp.dot(qb, wx_ref[...], preferred_element_type=jnp.float32)
    h = jnp.maximum(g - u[:, None, :] + b0_ref[...][None, None, :], 0.0)
    h = h.reshape(256 * S, 64)
    h = jnp.maximum(
        jnp.dot(h, w1_ref[...], preferred_element_type=jnp.float32)
        + b1_ref[...][None, :], 0.0)
    h = jnp.maximum(
        jnp.dot(h, w2_ref[...], preferred_element_type=jnp.float32)
        + b2_ref[...][None, :], 0.0)
    h = h.reshape(256, S, 128)
    nm = nm_ref[0][:, 0]
    out_ref[0] = jnp.max(h, axis=1) * nm[:, None]


def _mlp_pool(g4, new_xyz, nm, wxt, b0, w1t, b1, w2t, b2):
    return pl.pallas_call(
        _mlp_body,
        grid=(B, M // 256),
        in_specs=[
            pl.BlockSpec((1, 256, S, 64), lambda b, m: (b, m, 0, 0)),
            pl.BlockSpec((1, 256, 3), lambda b, m: (b, m, 0)),
            pl.BlockSpec((1, 256, 8), lambda b, m: (b, m, 0)),
            pl.BlockSpec((3, 64), lambda b, m: (0, 0)),
            pl.BlockSpec((64,), lambda b, m: (0,)),
            pl.BlockSpec((64, 64), lambda b, m: (0, 0)),
            pl.BlockSpec((64,), lambda b, m: (0,)),
            pl.BlockSpec((64, 128), lambda b, m: (0, 0)),
            pl.BlockSpec((128,), lambda b, m: (0,)),
        ],
        out_specs=pl.BlockSpec((1, 256, 128), lambda b, m: (b, m, 0)),
        out_shape=jax.ShapeDtypeStruct((B, M, 128), jnp.float32),
    )(g4, new_xyz, nm, wxt, b0, w1t, b1, w2t, b2)


def kernel(xyz, features, w0, b0, w1, b1, w2, b2):
    xt = jnp.transpose(xyz, (0, 2, 1))
    xc = xt.reshape(B, 3, 128, 128)
    new_xyz = _fps(xc)

    idx_g, nm = _ball_query(xt, new_xyz)

    ft = jnp.transpose(features, (0, 2, 1))
    wxt = jnp.transpose(w0[:, :3])
    wft = jnp.transpose(w0[:, 3:])
    t = _pre_transform(xyz, ft, wxt, wft)

    idx3 = idx_g.reshape(NW, CH, CW)
    g = _gather_rows(idx3, t.reshape(B * N, 64))
    g4 = g.reshape(B, M, S, 64)

    pooled = _mlp_pool(g4, new_xyz, nm, wxt, b0,
                       jnp.transpose(w1), b1, jnp.transpose(w2), b2)
    return new_xyz, jnp.transpose(pooled, (0, 2, 1))


# speedup vs baseline: 1.4984x; 1.1277x over previous
import functools

import jax
import jax.numpy as jnp
from jax import lax
from jax.experimental import pallas as pl
from jax.experimental.pallas import tpu as pltpu
from jax.experimental.pallas import tpu_sc as plsc

B = 4
N = 16384
M = 1024
S = 32
R2 = 0.25
NW = 32
CH = 32
CW = 128


def _fps_body(xs_ref, x_ref, out_ref):
    ii = (lax.broadcasted_iota(jnp.int32, (128, 128), 0) * 128
          + lax.broadcasted_iota(jnp.int32, (128, 128), 1))
    xs = [[x_ref[b, c] for c in range(3)] for b in range(B)]

    def body(i, st):
        dists, fs = st
        nd, nf = [], []
        for b in range(B):
            x0, x1, x2 = xs[b]
            f = fs[b]
            c0 = xs_ref[b * 3 * N + f]
            c1 = xs_ref[(b * 3 + 1) * N + f]
            c2 = xs_ref[(b * 3 + 2) * N + f]
            out_ref[b, 0, i] = c0
            out_ref[b, 1, i] = c1
            out_ref[b, 2, i] = c2
            d = (x0 - c0) ** 2 + (x1 - c1) ** 2 + (x2 - c2) ** 2
            db = jnp.minimum(dists[b], d)
            m = jnp.max(db)
            nf.append(jnp.min(jnp.where(db == m, ii, jnp.int32(N))))
            nd.append(db)
        return tuple(nd), tuple(nf)

    lax.fori_loop(
        0, M, body,
        (tuple(jnp.full((128, 128), 1e10, jnp.float32) for _ in range(B)),
         tuple(jnp.int32(0) for _ in range(B))))


def _fps(xc):
    out = pl.pallas_call(
        _fps_body,
        in_specs=[pl.BlockSpec(memory_space=pltpu.SMEM),
                  pl.BlockSpec(memory_space=pltpu.VMEM)],
        out_shape=jax.ShapeDtypeStruct((B, 3, M), jnp.float32),
        out_specs=pl.BlockSpec(memory_space=pltpu.SMEM),
    )(xc.reshape(-1), xc)
    return jnp.transpose(out, (0, 2, 1))


def _bq_body(xt_ref, q_ref, idx_ref, nm_ref, d_ref):
    b = pl.program_id(0)
    k = pl.program_id(2)
    base = b * N

    @pl.when(k == 0)
    def _init():
        X = xt_ref[0]
        Q = q_ref[0]
        x2 = jnp.sum(X * X, axis=0, keepdims=True)
        q2 = jnp.sum(Q * Q, axis=1, keepdims=True)
        dot = jnp.dot(Q, X, preferred_element_type=jnp.float32)
        dist2 = (q2 + x2) - 2.0 * dot
        d_ref[...] = jnp.where(dist2 <= R2, dist2, jnp.inf)

    D = d_ref[...]
    ii = lax.broadcasted_iota(jnp.int32, (128, N), 1)
    mv = jnp.min(D, axis=1, keepdims=True)
    pos = jnp.min(jnp.where(D == mv, ii, jnp.int32(N)), axis=1)
    valid = mv[:, 0] <= R2

    @pl.when(k == 0)
    def _nm():
        nm_ref[0] = jnp.broadcast_to(
            valid.astype(jnp.float32)[:, None], (128, 8))

    fb = jnp.where(k == 0, jnp.full((128,), base, jnp.int32),
                   idx_ref[0, :, 0])
    sel = jnp.where(valid, pos + base, fb)
    jj = lax.broadcasted_iota(jnp.int32, (128, S), 1)
    idx_ref[0] = jnp.where(jj == k, sel[:, None], idx_ref[0])
    d_ref[...] = jnp.where(ii == pos[:, None], jnp.inf, D)


def _ball_query(xt, new_xyz):
    return pl.pallas_call(
        _bq_body,
        grid=(B, M // 128, S),
        in_specs=[
            pl.BlockSpec((1, 3, N), lambda b, m, k: (b, 0, 0)),
            pl.BlockSpec((1, 128, 3), lambda b, m, k: (b, m, 0)),
        ],
        out_specs=[
            pl.BlockSpec((1, 128, S), lambda b, m, k: (b, m, 0)),
            pl.BlockSpec((1, 128, 8), lambda b, m, k: (b, m, 0)),
        ],
        out_shape=[
            jax.ShapeDtypeStruct((B, M, S), jnp.int32),
            jax.ShapeDtypeStruct((B, M, 8), jnp.float32),
        ],
        scratch_shapes=[pltpu.VMEM((128, N), jnp.float32)],
    )(xt, new_xyz)


def _t_body(x_ref, f_ref, wx_ref, wf_ref, t_ref):
    t_ref[0] = (
        jnp.dot(x_ref[0], wx_ref[...], preferred_element_type=jnp.float32)
        + jnp.dot(f_ref[0], wf_ref[...], preferred_element_type=jnp.float32))


def _pre_transform(xyz, ft, wxt, wft):
    return pl.pallas_call(
        _t_body,
        grid=(B,),
        in_specs=[
            pl.BlockSpec((1, N, 3), lambda b: (b, 0, 0)),
            pl.BlockSpec((1, N, 64), lambda b: (b, 0, 0)),
            pl.BlockSpec((3, 64), lambda b: (0, 0)),
            pl.BlockSpec((64, 64), lambda b: (0, 0)),
        ],
        out_specs=pl.BlockSpec((1, N, 64), lambda b: (b, 0, 0)),
        out_shape=jax.ShapeDtypeStruct((B, N, 64), jnp.float32),
    )(xyz, ft, wxt, wft)


def _sc_gather_body(idx_hbm, tab_hbm, out_hbm, idx_v, rows_v, sem):
    wid = lax.axis_index("s") * 2 + lax.axis_index("c")
    pltpu.sync_copy(idx_hbm.at[wid], idx_v)

    def body(j, carry):
        pltpu.async_copy(tab_hbm.at[idx_v.at[j]], rows_v, sem).wait()
        pltpu.sync_copy(rows_v, out_hbm.at[wid * CH + j])
        return carry

    lax.fori_loop(0, CH, body, 0)


def _gather_rows(idx3, tab):
    mesh = plsc.VectorSubcoreMesh(core_axis_name="c", subcore_axis_name="s")
    fn = pl.kernel(
        _sc_gather_body,
        mesh=mesh,
        compiler_params=pltpu.CompilerParams(use_tc_tiling_on_sc=False),
        out_type=jax.ShapeDtypeStruct((NW * CH, CW, 64), jnp.float32),
        scratch_types=[
            pltpu.VMEM((CH, CW), jnp.int32),
            pltpu.VMEM((CW, 64), jnp.float32),
            pltpu.SemaphoreType.DMA,
        ],
    )
    return fn(idx3, tab)


def _mlp_body(g_ref, q_ref, nm_ref, wx_ref, b0_ref, w1_ref, b1_ref,
              w2_ref, b2_ref, out_ref):
    g = g_ref[0]
    qb = q_ref[0]
    u = jnp.dot(qb, wx_ref[...], preferred_element_type=jnp.float32)
    h = jnp.maximum(g - u[:, None, :] + b0_ref[...][None, None, :], 0.0)
    h = h.reshape(256 * S, 64)
    h = jnp.maximum(
        jnp.dot(h, w1_ref[...], preferred_element_type=jnp.float32)
        + b1_ref[...][None, :], 0.0)
    h = jnp.maximum(
        jnp.dot(h, w2_ref[...], preferred_element_type=jnp.float32)
        + b2_ref[...][None, :], 0.0)
    h = h.reshape(256, S, 128)
    nm = nm_ref[0][:, 0]
    out_ref[0] = jnp.max(h, axis=1) * nm[:, None]


def _mlp_pool(g4, new_xyz, nm, wxt, b0, w1t, b1, w2t, b2):
    return pl.pallas_call(
        _mlp_body,
        grid=(B, M // 256),
        in_specs=[
            pl.BlockSpec((1, 256, S, 64), lambda b, m: (b, m, 0, 0)),
            pl.BlockSpec((1, 256, 3), lambda b, m: (b, m, 0)),
            pl.BlockSpec((1, 256, 8), lambda b, m: (b, m, 0)),
            pl.BlockSpec((3, 64), lambda b, m: (0, 0)),
            pl.BlockSpec((64,), lambda b, m: (0,)),
            pl.BlockSpec((64, 64), lambda b, m: (0, 0)),
            pl.BlockSpec((64,), lambda b, m: (0,)),
            pl.BlockSpec((64, 128), lambda b, m: (0, 0)),
            pl.BlockSpec((128,), lambda b, m: (0,)),
        ],
        out_specs=pl.BlockSpec((1, 256, 128), lambda b, m: (b, m, 0)),
        out_shape=jax.ShapeDtypeStruct((B, M, 128), jnp.float32),
    )(g4, new_xyz, nm, wxt, b0, w1t, b1, w2t, b2)


def kernel(xyz, features, w0, b0, w1, b1, w2, b2):
    xt = jnp.transpose(xyz, (0, 2, 1))
    xc = xt.reshape(B, 3, 128, 128)
    new_xyz = _fps(xc)

    idx_g, nm = _ball_query(xt, new_xyz)

    ft = jnp.transpose(features, (0, 2, 1))
    wxt = jnp.transpose(w0[:, :3])
    wft = jnp.transpose(w0[:, 3:])
    t = _pre_transform(xyz, ft, wxt, wft)

    idx3 = idx_g.reshape(NW, CH, CW)
    g = _gather_rows(idx3, t.reshape(B * N, 64))
    g4 = g.reshape(B, M, S, 64)

    pooled = _mlp_pool(g4, new_xyz, nm, wxt, b0,
                       jnp.transpose(w1), b1, jnp.transpose(w2), b2)
    return new_xyz, jnp.transpose(pooled, (0, 2, 1))
